# Initial kernel scaffold; baseline (speedup 1.0000x reference)
import jax
import jax.numpy as jnp
from jax.experimental import pallas as pl


def kernel(x, edge_index, edge_weight, alpha):
    raise NotImplementedError("write your pallas kernel here")


# speedup vs baseline: 3.4424x; 3.4424x over previous
import functools

import jax
import jax.numpy as jnp
from jax import lax
from jax.experimental import pallas as pl
from jax.experimental.pallas import tpu as pltpu
from jax.experimental.pallas import tpu_sc as plsc

_NC = 2
_NS = 16
_LANES = 16
_K = 128


def _sc_partials(src, dst, w, x):
    e_pad = src.shape[0]
    n, d = x.shape
    n_workers = _NC * _NS
    chunks = e_pad // (n_workers * _K)
    n_acc = ((n + _NS * _K - 1) // (_NS * _K)) * (_NS * _K)
    rows_per_tile = n_acc // _NS
    mesh = plsc.VectorSubcoreMesh(core_axis_name="c", subcore_axis_name="s")

    @functools.partial(
        pl.kernel,
        out_type=jax.ShapeDtypeStruct((_NC, n_acc, d), jnp.float32),
        mesh=mesh,
        scratch_types=[
            pltpu.VMEM((_K,), jnp.int32),
            pltpu.VMEM((_K,), jnp.int32),
            pltpu.VMEM((_K,), jnp.float32),
            pltpu.VMEM((_K, d), jnp.float32),
            pltpu.VMEM_SHARED((n_acc, d), jnp.float32),
            pltpu.SemaphoreType.DMA,
        ],
    )
    def k(src_hbm, dst_hbm, w_hbm, x_hbm, out_hbm, srcv, dstv, wv,
          rows, acc, sem):
        cid = lax.axis_index("c")
        sid = lax.axis_index("s")
        wid = cid * _NS + sid

        zeros16 = jnp.zeros((_LANES,), jnp.float32)

        def zrow(r, carry):
            for j in range(d // _LANES):
                rows[r, pl.ds(j * _LANES, _LANES)] = zeros16
            return carry

        lax.fori_loop(0, _K, zrow, 0)
        for i in range(rows_per_tile // _K):
            pltpu.sync_copy(
                rows, acc.at[pl.ds(sid * rows_per_tile + i * _K, _K)])
        plsc.subcore_barrier()

        base0 = wid * chunks * _K

        def chunk_body(t, carry):
            base = base0 + t * _K
            pltpu.sync_copy(src_hbm.at[pl.ds(base, _K)], srcv)
            pltpu.sync_copy(dst_hbm.at[pl.ds(base, _K)], dstv)
            pltpu.sync_copy(w_hbm.at[pl.ds(base, _K)], wv)
            pltpu.async_copy(x_hbm.at[srcv], rows, sem).wait()

            def scale(g, c2):
                wvec = wv[pl.ds(g * _LANES, _LANES)]
                for i in range(_LANES):
                    ws = wvec[i]
                    eb = g * _LANES + i
                    for j in range(d // _LANES):
                        sl = pl.ds(j * _LANES, _LANES)
                        rows[eb, sl] = rows[eb, sl] * ws
                return c2

            lax.fori_loop(0, _K // _LANES, scale, 0)
            pltpu.sync_copy(rows, acc.at[dstv], add=True)
            return carry

        lax.fori_loop(0, chunks, chunk_body, 0)

        plsc.subcore_barrier()
        pltpu.sync_copy(
            acc.at[pl.ds(sid * rows_per_tile, rows_per_tile)],
            out_hbm.at[cid, pl.ds(sid * rows_per_tile, rows_per_tile)])

    return k(src, dst, w, x)


def _mix(x, p0, p1, alpha):
    n, d = x.shape
    blk = 1000

    def body(a_ref, x_ref, p0_ref, p1_ref, o_ref):
        a = a_ref[0]
        o_ref[...] = a * x_ref[...] + (1.0 - a) * (p0_ref[...] + p1_ref[...])

    return pl.pallas_call(
        body,
        grid=(n // blk,),
        in_specs=[
            pl.BlockSpec(memory_space=pltpu.SMEM),
            pl.BlockSpec((blk, d), lambda i: (i, 0)),
            pl.BlockSpec((blk, d), lambda i: (i, 0)),
            pl.BlockSpec((blk, d), lambda i: (i, 0)),
        ],
        out_specs=pl.BlockSpec((blk, d), lambda i: (i, 0)),
        out_shape=jax.ShapeDtypeStruct((n, d), jnp.float32),
    )(alpha, x, p0, p1)


def kernel(x, edge_index, edge_weight, alpha):
    n, d = x.shape
    e = edge_weight.shape[0]
    per = _NC * _NS * _K
    e_pad = ((e + per - 1) // per) * per
    pad = e_pad - e
    src = edge_index[1].astype(jnp.int32)
    dst = edge_index[0].astype(jnp.int32)
    w = edge_weight.astype(jnp.float32)
    if pad:
        src = jnp.concatenate([src, jnp.zeros((pad,), jnp.int32)])
        dst = jnp.concatenate([dst, jnp.zeros((pad,), jnp.int32)])
        w = jnp.concatenate([w, jnp.zeros((pad,), jnp.float32)])
    parts = _sc_partials(src, dst, w, x)
    return _mix(x, parts[0, :n], parts[1, :n], alpha.astype(jnp.float32))
